# Initial kernel scaffold; baseline (speedup 1.0000x reference)
import jax
import jax.numpy as jnp
from jax.experimental import pallas as pl


def kernel(x, edge_index, edge_attr, W0, b0, W1, b1, Wc, bc):
    raise NotImplementedError("write your pallas kernel here")


# speedup vs baseline: 9.4835x; 9.4835x over previous
import functools

import jax
import jax.numpy as jnp
from jax import lax
from jax.experimental import pallas as pl
from jax.experimental.pallas import tpu as pltpu
from jax.experimental.pallas import tpu_sc as plsc

F32 = jnp.float32
I32 = jnp.int32

NC = 2
NS = 16
NW = NC * NS


def _pad_rows(N):
    q = 16 * 16
    return ((N + q - 1) // q) * q


def _make_deg(N, E, interpret=False):
    EPT = E // NW
    K = 80
    NB = EPT // K
    assert EPT * NW == E and NB * K == EPT
    NP = _pad_rows(N)
    ZS = NP // NS
    assert ZS % 16 == 0

    mesh = plsc.VectorSubcoreMesh(core_axis_name="c", subcore_axis_name="s")

    @functools.partial(
        pl.kernel,
        out_type=jax.ShapeDtypeStruct((NC * NP,), F32),
        mesh=mesh,
        interpret=interpret,
        scratch_types=[
            pltpu.VMEM((K,), I32),
            pltpu.VMEM((K,), F32),
            pltpu.VMEM((ZS,), F32),
            pltpu.VMEM_SHARED((NP,), F32),
        ],
    )
    def deg_kernel(dst_hbm, ew_hbm, out_hbm, idx_v, w_v, zb_v, deg_sh):
        c = lax.axis_index("c")
        s = lax.axis_index("s")

        def zb(i, carry):
            zb_v[pl.ds(i * 16, 16)] = jnp.zeros((16,), F32)
            return carry

        lax.fori_loop(0, ZS // 16, zb, 0)
        pltpu.sync_copy(zb_v, deg_sh.at[pl.ds(s * ZS, ZS)])
        plsc.subcore_barrier()

        wid = c * NS + s

        def body(it, carry):
            base = wid * EPT + it * K
            pltpu.sync_copy(dst_hbm.at[pl.ds(base, K)], idx_v)
            pltpu.sync_copy(ew_hbm.at[pl.ds(base, K)], w_v)
            pltpu.sync_copy(w_v, deg_sh.at[idx_v], add=True)
            return carry

        lax.fori_loop(0, NB, body, 0)
        plsc.subcore_barrier()
        pltpu.sync_copy(deg_sh.at[pl.ds(s * ZS, ZS)],
                        out_hbm.at[pl.ds(c * NP + s * ZS, ZS)])

    return deg_kernel


def _make_agg(N, D, E, interpret=False):
    EPT = E // NW
    K = 80
    NB = EPT // K
    assert EPT * NW == E and NB * K == EPT and D == 128
    NP = _pad_rows(N)
    SR = NP // NS
    ZR = 128
    NZ = SR // ZR
    assert NZ * ZR == SR

    mesh = plsc.VectorSubcoreMesh(core_axis_name="c", subcore_axis_name="s")

    @functools.partial(
        pl.kernel,
        out_type=jax.ShapeDtypeStruct((NC, NP, D), F32),
        mesh=mesh,
        interpret=interpret,
        scratch_types=[
            pltpu.VMEM((K,), I32),
            pltpu.VMEM((K,), I32),
            pltpu.VMEM((K,), F32),
            pltpu.VMEM((K, D), F32),
            pltpu.VMEM((ZR, D), F32),
            pltpu.VMEM_SHARED((NP, D), F32),
            pltpu.SemaphoreType.DMA,
        ],
    )
    def agg_kernel(g_hbm, src_hbm, dst_hbm, ew_hbm, out_hbm,
                   src_v, dst_v, ew_v, rows_v, zb_v, acc_sh, sem):
        c = lax.axis_index("c")
        s = lax.axis_index("s")

        def zb(i, carry):
            for j in range(D // 16):
                zb_v[i, pl.ds(j * 16, 16)] = jnp.zeros((16,), F32)
            return carry

        lax.fori_loop(0, ZR, zb, 0)
        for k in range(NZ):
            pltpu.sync_copy(zb_v, acc_sh.at[pl.ds(s * SR + k * ZR, ZR)])
        plsc.subcore_barrier()

        wid = c * NS + s

        def body(it, carry):
            base = wid * EPT + it * K
            pltpu.sync_copy(src_hbm.at[pl.ds(base, K)], src_v)
            pltpu.sync_copy(dst_hbm.at[pl.ds(base, K)], dst_v)
            pltpu.sync_copy(ew_hbm.at[pl.ds(base, K)], ew_v)
            pltpu.async_copy(g_hbm.at[src_v], rows_v, sem).wait()

            def mul(grp, c2):
                ewv = ew_v[pl.ds(grp * 16, 16)]
                for l in range(16):
                    w = ewv[l]
                    i = grp * 16 + l
                    for j in range(D // 16):
                        sl = pl.ds(j * 16, 16)
                        rows_v[i, sl] = rows_v[i, sl] * w
                return c2

            lax.fori_loop(0, K // 16, mul, 0)
            pltpu.sync_copy(rows_v, acc_sh.at[dst_v], add=True)
            return carry

        lax.fori_loop(0, NB, body, 0)
        plsc.subcore_barrier()
        for k in range(NZ):
            sl = pl.ds(s * SR + k * ZR, ZR)
            pltpu.sync_copy(acc_sh.at[sl], out_hbm.at[c, sl])

    return agg_kernel


_BM = 400


def _tc0_body(deg0, deg1, x, w, dinvb_out, g_out):
    d = deg0[...] + deg1[...] + 1.0
    dinv = jnp.where(d > 0, lax.rsqrt(d), 0.0)
    db = jnp.broadcast_to(dinv, (_BM, x.shape[1]))
    dinvb_out[...] = db
    h = jnp.dot(x[...], w[...], preferred_element_type=F32)
    g_out[...] = h * db


def _tc_mid_body(p0, p1, g, db, b, w, g_out):
    a = jnp.maximum(db[...] * (p0[...] + p1[...] + g[...]) + b[...], 0.0)
    h = jnp.dot(a, w[...], preferred_element_type=F32)
    g_out[...] = h * db[...]


def _tc_fin_body(p0, p1, g, db, b, wc, bc, out):
    a = jnp.maximum(db[...] * (p0[...] + p1[...] + g[...]) + b[...], 0.0)
    out[...] = jnp.dot(a, wc[...], preferred_element_type=F32) + bc[...]


def _row_spec(D):
    return pl.BlockSpec((_BM, D), lambda i: (i, 0))


def _full_spec(shape):
    return pl.BlockSpec(shape, lambda i: (0,) * len(shape))


def kernel(x, edge_index, edge_attr, W0, b0, W1, b1, Wc, bc):
    N, D = x.shape
    E = edge_attr.shape[0]
    C = Wc.shape[1]
    NP = _pad_rows(N)
    assert N % _BM == 0
    grid = (N // _BM,)

    src = edge_index[0]
    dst = edge_index[1]

    deg_fn = _make_deg(N, E)
    agg_fn = _make_agg(N, D, E)

    degp = deg_fn(dst, edge_attr).reshape(NC, NP)
    deg0 = degp[0, :N].reshape(N, 1)
    deg1 = degp[1, :N].reshape(N, 1)

    dinvb, g0 = pl.pallas_call(
        _tc0_body,
        grid=grid,
        in_specs=[_row_spec(1), _row_spec(1), _row_spec(D), _full_spec((D, D))],
        out_specs=[_row_spec(D), _row_spec(D)],
        out_shape=[jax.ShapeDtypeStruct((N, D), F32),
                   jax.ShapeDtypeStruct((N, D), F32)],
    )(deg0, deg1, x, W0)

    p = agg_fn(g0, src, dst, edge_attr)

    g1 = pl.pallas_call(
        _tc_mid_body,
        grid=grid,
        in_specs=[_row_spec(D), _row_spec(D), _row_spec(D), _row_spec(D),
                  _full_spec((1, D)), _full_spec((D, D))],
        out_specs=_row_spec(D),
        out_shape=jax.ShapeDtypeStruct((N, D), F32),
    )(p[0, :N], p[1, :N], g0, dinvb, b0.reshape(1, D), W1)

    p2 = agg_fn(g1, src, dst, edge_attr)

    out = pl.pallas_call(
        _tc_fin_body,
        grid=grid,
        in_specs=[_row_spec(D), _row_spec(D), _row_spec(D), _row_spec(D),
                  _full_spec((1, D)), _full_spec((D, C)), _full_spec((1, C))],
        out_specs=pl.BlockSpec((_BM, C), lambda i: (i, 0)),
        out_shape=jax.ShapeDtypeStruct((N, C), F32),
    )(p2[0, :N], p2[1, :N], g1, dinvb, b1.reshape(1, D), Wc, bc.reshape(1, C))

    return out
